# baseline (speedup 1.0000x reference)
import functools

import jax
import jax.numpy as jnp
from jax import lax
from jax.experimental import pallas as pl
from jax.experimental.pallas import tpu as pltpu
from jax.experimental.pallas import tpu_sc as plsc

_CHUNK = 128
_L = 16
_TC_ROWS = 8192
_TC_BLOCK = 1024


@functools.cache
def _build_sc(B, R, V, D, NC, NS):
    NW = NC * NS
    b_per_w = (B - R) // NW
    n_ch = b_per_w // _CHUNK
    row0 = R // _CHUNK
    V_pad = -(-V // _L) * _L
    n_stage = V_pad // _L
    mesh = plsc.VectorSubcoreMesh(core_axis_name="c", subcore_axis_name="s")

    @functools.partial(
        pl.kernel,
        mesh=mesh,
        out_type=jax.ShapeDtypeStruct((B, D), jnp.float32),
        scratch_types=[
            pltpu.VMEM((n_ch, _CHUNK), jnp.int32),
            pltpu.VMEM((b_per_w, D), jnp.float32),
            pltpu.VMEM((_L, D), jnp.float32),
            pltpu.VMEM_SHARED((V_pad, D), jnp.float32),
            pltpu.SemaphoreType.DMA,
            pltpu.SemaphoreType.DMA,
        ],
    )
    def k(labels_hbm, table_hbm, out_hbm, idx_v, rows_v, stage_v, table_sh,
          gsem, wsem):
        cid = lax.axis_index("c")
        sid = lax.axis_index("s")
        wid = sid * NC + cid
        base = R + wid * b_per_w

        @pl.when(sid < n_stage)
        def _():
            ridx = jnp.minimum(sid * _L + lax.iota(jnp.int32, _L), V - 1)
            pltpu.async_copy(table_hbm.at[ridx], stage_v, gsem).wait()
            pltpu.sync_copy(stage_v, table_sh.at[pl.ds(sid * _L, _L)])

        pltpu.sync_copy(
            labels_hbm.at[pl.ds(row0 + wid * n_ch, n_ch)], idx_v
        )
        plsc.subcore_barrier()

        gathers = []
        for j in range(n_ch):
            gathers.append(
                pltpu.async_copy(
                    table_sh.at[idx_v.at[j]],
                    rows_v.at[pl.ds(j * _CHUNK, _CHUNK)],
                    gsem,
                )
            )
        writes = []
        for j in range(n_ch):
            gathers[j].wait()
            writes.append(
                pltpu.async_copy(
                    rows_v.at[pl.ds(j * _CHUNK, _CHUNK)],
                    out_hbm.at[pl.ds(base + j * _CHUNK, _CHUNK)],
                    wsem,
                )
            )
        for w in writes:
            w.wait()

    return k


def _tc_body(out_alias_ref, labels_ref, table_ref, out_ref):
    del out_alias_ref
    lbl = labels_ref[0, 0, :]
    onehot = (
        lbl[:, None] == lax.broadcasted_iota(jnp.int32, (_TC_BLOCK, 128), 1)
    ).astype(jnp.float32)
    out_ref[...] = jnp.dot(
        onehot, table_ref[...], preferred_element_type=jnp.float32
    )


@functools.cache
def _build_tc(B, R, D):
    return pl.pallas_call(
        _tc_body,
        grid=(R // _TC_BLOCK,),
        in_specs=[
            pl.BlockSpec(memory_space=pl.ANY),
            pl.BlockSpec((1, 1, _TC_BLOCK), lambda i: (i, 0, 0)),
            pl.BlockSpec((128, D), lambda i: (0, 0)),
        ],
        out_specs=pl.BlockSpec((_TC_BLOCK, D), lambda i: (i, 0)),
        out_shape=jax.ShapeDtypeStruct((B, D), jnp.float32),
        input_output_aliases={0: 0},
    )


def kernel(labels, table):
    (B,) = labels.shape
    V, D = table.shape
    info = plsc.get_sparse_core_info()
    labels_i = labels.astype(jnp.int32)
    labels2d = labels_i.reshape(B // _CHUNK, _CHUNK)
    R = _TC_ROWS
    sc_out = _build_sc(B, R, V, D, info.num_cores, info.num_subcores)(
        labels2d, table
    )
    labels_tc = labels_i[:R].reshape(R // _TC_BLOCK, 1, _TC_BLOCK)
    table128 = jnp.pad(table, ((0, 128 - V), (0, 0)))
    return _build_tc(B, R, D)(sc_out, labels_tc, table128)


# speedup vs baseline: 1.2403x; 1.2403x over previous
import functools

import jax
import jax.numpy as jnp
from jax import lax
from jax.experimental import pallas as pl
from jax.experimental.pallas import tpu as pltpu
from jax.experimental.pallas import tpu_sc as plsc

_CHUNK = 128


@functools.cache
def _build(B, V, D, NC, NS):
    NW = NC * NS
    b_per_w = B // NW
    n_ch = b_per_w // _CHUNK
    mesh = plsc.VectorSubcoreMesh(core_axis_name="c", subcore_axis_name="s")

    @functools.partial(
        pl.kernel,
        mesh=mesh,
        out_type=jax.ShapeDtypeStruct((B, D), jnp.float32),
        scratch_types=[
            pltpu.VMEM((n_ch, _CHUNK), jnp.int32),
            pltpu.VMEM((b_per_w, D), jnp.float32),
            pltpu.VMEM_SHARED((V, D), jnp.float32),
            pltpu.SemaphoreType.DMA,
            pltpu.SemaphoreType.DMA,
        ],
    )
    def k(labels_hbm, table_hbm, out_hbm, idx_v, rows_v, table_sh, gsem,
          wsem):
        cid = lax.axis_index("c")
        sid = lax.axis_index("s")
        wid = sid * NC + cid
        base = wid * b_per_w

        idx_cp = pltpu.async_copy(
            labels_hbm.at[pl.ds(wid * n_ch, n_ch)], idx_v, wsem
        )

        @pl.when(sid == 0)
        def _():
            pltpu.sync_copy(table_hbm, table_sh)

        idx_cp.wait()
        plsc.subcore_barrier()

        gathers = []
        for j in range(n_ch):
            gathers.append(
                pltpu.async_copy(
                    table_sh.at[idx_v.at[j]],
                    rows_v.at[pl.ds(j * _CHUNK, _CHUNK)],
                    gsem,
                )
            )
        writes = []
        for j in range(n_ch):
            gathers[j].wait()
            writes.append(
                pltpu.async_copy(
                    rows_v.at[pl.ds(j * _CHUNK, _CHUNK)],
                    out_hbm.at[pl.ds(base + j * _CHUNK, _CHUNK)],
                    wsem,
                )
            )
        for w in writes:
            w.wait()

    return k


def kernel(labels, table):
    (B,) = labels.shape
    V, D = table.shape
    info = plsc.get_sparse_core_info()
    labels2d = labels.astype(jnp.int32).reshape(B // _CHUNK, _CHUNK)
    return _build(B, V, D, info.num_cores, info.num_subcores)(labels2d, table)
